# baseline (speedup 1.0000x reference)
import functools

import jax
import jax.numpy as jnp
from jax import lax
from jax.experimental import pallas as pl
from jax.experimental.pallas import tpu as pltpu
from jax.experimental.pallas import tpu_sc as plsc

N = 10000
E = 320000
D = 128

NC = 2
NS = 16

CHUNK = 128
ROWS_PER_TILE = 640
N_PAD = NS * ROWS_PER_TILE
C0 = 80
C1 = 80
TOTAL_CHUNKS = NS * (C0 + C1)
E_PAD = TOTAL_CHUNKS * CHUNK


def _sc_aggregate(x, src, dst, z2d, z1d, ones_h):
  mesh = plsc.VectorSubcoreMesh(core_axis_name="c", subcore_axis_name="s")

  @functools.partial(
      pl.kernel,
      out_type=[
          jax.ShapeDtypeStruct((NC, N_PAD, D), jnp.float32),
          jax.ShapeDtypeStruct((NC, N_PAD), jnp.float32),
      ],
      mesh=mesh,
      scratch_types=[
          pltpu.VMEM((CHUNK,), jnp.int32),
          pltpu.VMEM((CHUNK,), jnp.int32),
          pltpu.VMEM((CHUNK,), jnp.float32),
          pltpu.VMEM((CHUNK, D), jnp.float32),
          pltpu.VMEM_SHARED((N_PAD, D), jnp.float32),
          pltpu.VMEM_SHARED((N_PAD,), jnp.float32),
          pltpu.SemaphoreType.DMA,
      ],
  )
  def body(x_h, src_h, dst_h, z2d_h, z1d_h, ones_hbm, agg_out, cnt_out,
           src_v, dst_v, ones_v, rows_v, agg_sh, cnt_sh, sem):
    cid = lax.axis_index("c")
    sid = lax.axis_index("s")

    row0 = sid * ROWS_PER_TILE
    pltpu.sync_copy(z2d_h, agg_sh.at[pl.ds(row0, ROWS_PER_TILE)])
    pltpu.sync_copy(z1d_h, cnt_sh.at[pl.ds(row0, ROWS_PER_TILE)])
    pltpu.sync_copy(ones_hbm, ones_v)
    plsc.subcore_barrier()

    def run_core(n_chunks, base_edge):
      def step(c, carry):
        off = pl.multiple_of(base_edge + c * CHUNK, CHUNK)
        pltpu.sync_copy(src_h.at[pl.ds(off, CHUNK)], src_v)
        pltpu.sync_copy(dst_h.at[pl.ds(off, CHUNK)], dst_v)
        pltpu.async_copy(x_h.at[src_v], rows_v, sem).wait()
        pltpu.sync_copy(rows_v, agg_sh.at[dst_v], add=True)
        pltpu.sync_copy(ones_v, cnt_sh.at[dst_v], add=True)
        return carry

      lax.fori_loop(0, n_chunks, step, 0)

    @pl.when(cid == 0)
    def _():
      run_core(C0, sid * C0 * CHUNK)

    @pl.when(cid == 1)
    def _():
      run_core(C1, (NS * C0 + sid * C1) * CHUNK)

    plsc.subcore_barrier()

    pltpu.sync_copy(agg_sh.at[pl.ds(row0, ROWS_PER_TILE)],
                    agg_out.at[cid].at[pl.ds(row0, ROWS_PER_TILE)])
    pltpu.sync_copy(cnt_sh.at[pl.ds(row0, ROWS_PER_TILE)],
                    cnt_out.at[cid].at[pl.ds(row0, ROWS_PER_TILE)])

  return body(x, src, dst, z2d, z1d, ones_h)


ROW_BLK = 2000


def _tc_body(x_ref, agg_ref, cnt_ref, wl_ref, wr_ref, b_ref, out_ref):
  agg = agg_ref[0] + agg_ref[1]
  cnt = cnt_ref[0] + cnt_ref[1]
  mean = agg * (1.0 / jnp.maximum(cnt, 1.0))
  acc = jnp.dot(mean, wl_ref[...], preferred_element_type=jnp.float32)
  acc = acc + jnp.dot(x_ref[...], wr_ref[...],
                      preferred_element_type=jnp.float32)
  acc = acc + b_ref[...]
  out_ref[...] = jnp.maximum(acc, 0.0)


def _tc_tail(x, agg, cnt, wl_t, wr_t, b2d):
  grid = N // ROW_BLK
  return pl.pallas_call(
      _tc_body,
      grid=(grid,),
      in_specs=[
          pl.BlockSpec((ROW_BLK, D), lambda i: (i, 0)),
          pl.BlockSpec((NC, ROW_BLK, D), lambda i: (0, i, 0)),
          pl.BlockSpec((NC, ROW_BLK, 1), lambda i: (0, i, 0)),
          pl.BlockSpec((D, D), lambda i: (0, 0)),
          pl.BlockSpec((D, D), lambda i: (0, 0)),
          pl.BlockSpec((1, D), lambda i: (0, 0)),
      ],
      out_specs=pl.BlockSpec((ROW_BLK, D), lambda i: (i, 0)),
      out_shape=jax.ShapeDtypeStruct((N, D), jnp.float32),
  )(x, agg, cnt, wl_t, wr_t, b2d)


@jax.jit
def kernel(x, edge_index, W_l, b_l, W_r):
  src = edge_index[0].astype(jnp.int32)
  dst = edge_index[1].astype(jnp.int32)
  pad = E_PAD - E
  src = jnp.concatenate([src, jnp.zeros((pad,), jnp.int32)])
  dst = jnp.concatenate([dst, jnp.full((pad,), N, jnp.int32)])
  half = NS * C0 * CHUNK
  src = jnp.concatenate([src[:half], src[half:] + N])

  z2d = jnp.zeros((ROWS_PER_TILE, D), jnp.float32)
  z1d = jnp.zeros((ROWS_PER_TILE,), jnp.float32)
  ones_h = jnp.ones((CHUNK,), jnp.float32)

  xx = jnp.concatenate([x, x], axis=0)
  agg, cnt = _sc_aggregate(xx, src, dst, z2d, z1d, ones_h)

  out = _tc_tail(x, agg, cnt.reshape(NC, N_PAD, 1),
                 W_l.T, W_r.T, b_l.reshape(1, D))
  return out


# speedup vs baseline: 1.1908x; 1.1908x over previous
import functools

import jax
import jax.numpy as jnp
from jax import lax
from jax.experimental import pallas as pl
from jax.experimental.pallas import tpu as pltpu
from jax.experimental.pallas import tpu_sc as plsc

N = 10000
E = 320000
D = 128

NC = 2
NS = 16

CHUNK = 128
ROWS_PER_TILE = 640
N_PAD = NS * ROWS_PER_TILE
C0 = 80
C1 = 80
TOTAL_CHUNKS = NS * (C0 + C1)
E_PAD = TOTAL_CHUNKS * CHUNK


def _sc_aggregate(x, src, dst, z2d, z1d, ones_h):
  mesh = plsc.VectorSubcoreMesh(core_axis_name="c", subcore_axis_name="s")

  @functools.partial(
      pl.kernel,
      out_type=[
          jax.ShapeDtypeStruct((NC, N_PAD, D), jnp.float32),
          jax.ShapeDtypeStruct((NC, N_PAD), jnp.float32),
      ],
      mesh=mesh,
      scratch_types=[
          pltpu.VMEM((CHUNK,), jnp.int32),
          pltpu.VMEM((CHUNK,), jnp.int32),
          pltpu.VMEM((CHUNK,), jnp.float32),
          pltpu.VMEM((CHUNK, D), jnp.float32),
          pltpu.VMEM_SHARED((N_PAD, D), jnp.float32),
          pltpu.VMEM_SHARED((N_PAD,), jnp.float32),
          pltpu.SemaphoreType.DMA,
      ],
  )
  def body(x_h, src_h, dst_h, z2d_h, z1d_h, ones_hbm, agg_out, cnt_out,
           src_v, dst_v, ones_v, rows_v, agg_sh, cnt_sh, sem):
    cid = lax.axis_index("c")
    sid = lax.axis_index("s")

    row0 = sid * ROWS_PER_TILE
    pltpu.sync_copy(z2d_h, agg_sh.at[pl.ds(row0, ROWS_PER_TILE)])
    pltpu.sync_copy(z1d_h, cnt_sh.at[pl.ds(row0, ROWS_PER_TILE)])
    pltpu.sync_copy(ones_hbm, ones_v)
    plsc.subcore_barrier()

    def run_core(n_chunks, base_edge):
      def step(c, carry):
        off = pl.multiple_of(base_edge + c * CHUNK, CHUNK)
        pltpu.sync_copy(src_h.at[pl.ds(off, CHUNK)], src_v)
        pltpu.sync_copy(dst_h.at[pl.ds(off, CHUNK)], dst_v)
        pltpu.async_copy(x_h.at[src_v], rows_v, sem).wait()
        pltpu.sync_copy(rows_v, agg_sh.at[dst_v], add=True)
        pltpu.sync_copy(ones_v, cnt_sh.at[dst_v], add=True)
        return carry

      lax.fori_loop(0, n_chunks, step, 0)

    @pl.when(cid == 0)
    def _():
      run_core(C0, sid * C0 * CHUNK)

    @pl.when(cid == 1)
    def _():
      run_core(C1, (NS * C0 + sid * C1) * CHUNK)

    plsc.subcore_barrier()

    pltpu.sync_copy(agg_sh.at[pl.ds(row0, ROWS_PER_TILE)],
                    agg_out.at[cid].at[pl.ds(row0, ROWS_PER_TILE)])
    pltpu.sync_copy(cnt_sh.at[pl.ds(row0, ROWS_PER_TILE)],
                    cnt_out.at[cid].at[pl.ds(row0, ROWS_PER_TILE)])

  return body(x, src, dst, z2d, z1d, ones_h)


ROW_BLK = 2000


def _tc_body(x_ref, agg_ref, cnt_ref, wl_ref, wr_ref, b_ref, out_ref):
  agg = agg_ref[0] + agg_ref[1]
  cnt = cnt_ref[0] + cnt_ref[1]
  mean = agg * (1.0 / jnp.maximum(cnt, 1.0))
  acc = jnp.dot(mean, wl_ref[...], preferred_element_type=jnp.float32)
  acc = acc + jnp.dot(x_ref[...], wr_ref[...],
                      preferred_element_type=jnp.float32)
  acc = acc + b_ref[...]
  out_ref[...] = jnp.maximum(acc, 0.0)


def _tc_tail(x, agg, cnt, wl_t, wr_t, b2d):
  grid = N // ROW_BLK
  return pl.pallas_call(
      _tc_body,
      grid=(grid,),
      in_specs=[
          pl.BlockSpec((ROW_BLK, D), lambda i: (i, 0)),
          pl.BlockSpec((NC, ROW_BLK, D), lambda i: (0, i, 0)),
          pl.BlockSpec((NC, ROW_BLK, 1), lambda i: (0, i, 0)),
          pl.BlockSpec((D, D), lambda i: (0, 0)),
          pl.BlockSpec((D, D), lambda i: (0, 0)),
          pl.BlockSpec((1, D), lambda i: (0, 0)),
      ],
      out_specs=pl.BlockSpec((ROW_BLK, D), lambda i: (i, 0)),
      out_shape=jax.ShapeDtypeStruct((N, D), jnp.float32),
  )(x, agg, cnt, wl_t, wr_t, b2d)


@jax.jit
def kernel(x, edge_index, W_l, b_l, W_r):
  src = edge_index[0].astype(jnp.int32)
  dst = edge_index[1].astype(jnp.int32)
  pad = E_PAD - E
  src = jnp.concatenate([src, jnp.zeros((pad,), jnp.int32)])
  dst = jnp.concatenate([dst, jnp.full((pad,), N, jnp.int32)])

  z2d = jnp.zeros((ROWS_PER_TILE, D), jnp.float32)
  z1d = jnp.zeros((ROWS_PER_TILE,), jnp.float32)
  ones_h = jnp.ones((CHUNK,), jnp.float32)

  agg, cnt = _sc_aggregate(x, src, dst, z2d, z1d, ones_h)

  out = _tc_tail(x, agg, cnt.reshape(NC, N_PAD, 1),
                 W_l.T, W_r.T, b_l.reshape(1, D))
  return out


# speedup vs baseline: 1.6115x; 1.3534x over previous
import functools

import jax
import jax.numpy as jnp
from jax import lax
from jax.experimental import pallas as pl
from jax.experimental.pallas import tpu as pltpu
from jax.experimental.pallas import tpu_sc as plsc

N = 10000
E = 320000
D = 128

NC = 2
NS = 16
NW = NC * NS

CHUNK = 128
ROWS_PER_TILE = 640
N_PAD = NS * ROWS_PER_TILE
EDGES_PER_TILE = ((E + NW * CHUNK - 1) // (NW * CHUNK)) * CHUNK
E_PAD = EDGES_PER_TILE * NW
N_CHUNKS = EDGES_PER_TILE // CHUNK


def _sc_aggregate(x, src, dst, z2d, z1d, ones_h):
  mesh = plsc.VectorSubcoreMesh(core_axis_name="c", subcore_axis_name="s")

  @functools.partial(
      pl.kernel,
      out_type=[
          jax.ShapeDtypeStruct((NC, N_PAD, D), jnp.float32),
          jax.ShapeDtypeStruct((NC, N_PAD), jnp.float32),
      ],
      mesh=mesh,
      scratch_types=[
          pltpu.VMEM((CHUNK,), jnp.int32),
          pltpu.VMEM((CHUNK,), jnp.int32),
          pltpu.VMEM((CHUNK,), jnp.float32),
          pltpu.VMEM((CHUNK, D), jnp.float32),
          pltpu.VMEM_SHARED((N_PAD, D), jnp.float32),
          pltpu.VMEM_SHARED((N_PAD,), jnp.float32),
          pltpu.SemaphoreType.DMA,
      ],
  )
  def body(x_h, src_h, dst_h, z2d_h, z1d_h, ones_hbm, agg_out, cnt_out,
           src_v, dst_v, ones_v, rows_v, agg_sh, cnt_sh, sem):
    cid = lax.axis_index("c")
    sid = lax.axis_index("s")
    wid = cid * NS + sid

    row0 = sid * ROWS_PER_TILE
    pltpu.sync_copy(z2d_h, agg_sh.at[pl.ds(row0, ROWS_PER_TILE)])
    pltpu.sync_copy(z1d_h, cnt_sh.at[pl.ds(row0, ROWS_PER_TILE)])
    pltpu.sync_copy(ones_hbm, ones_v)
    plsc.subcore_barrier()

    base = wid * EDGES_PER_TILE

    def step(c, carry):
      off = pl.multiple_of(base + c * CHUNK, CHUNK)
      pltpu.sync_copy(src_h.at[pl.ds(off, CHUNK)], src_v)
      pltpu.sync_copy(dst_h.at[pl.ds(off, CHUNK)], dst_v)
      pltpu.async_copy(x_h.at[src_v], rows_v, sem).wait()
      pltpu.sync_copy(rows_v, agg_sh.at[dst_v], add=True)
      pltpu.sync_copy(ones_v, cnt_sh.at[dst_v], add=True)
      return carry

    lax.fori_loop(0, N_CHUNKS, step, 0)
    plsc.subcore_barrier()

    pltpu.sync_copy(agg_sh.at[pl.ds(row0, ROWS_PER_TILE)],
                    agg_out.at[cid].at[pl.ds(row0, ROWS_PER_TILE)])
    pltpu.sync_copy(cnt_sh.at[pl.ds(row0, ROWS_PER_TILE)],
                    cnt_out.at[cid].at[pl.ds(row0, ROWS_PER_TILE)])

  return body(x, src, dst, z2d, z1d, ones_h)


ROW_BLK = 2000


def _tc_body(x_ref, agg_ref, cnt_ref, wl_ref, wr_ref, b_ref, out_ref):
  agg = agg_ref[0] + agg_ref[1]
  cnt = cnt_ref[0] + cnt_ref[1]
  mean = agg * (1.0 / jnp.maximum(cnt, 1.0))
  acc = jnp.dot(mean, wl_ref[...], preferred_element_type=jnp.float32)
  acc = acc + jnp.dot(x_ref[...], wr_ref[...],
                      preferred_element_type=jnp.float32)
  acc = acc + b_ref[...]
  out_ref[...] = jnp.maximum(acc, 0.0)


def _tc_tail(x, agg, cnt, wl_t, wr_t, b2d):
  grid = N // ROW_BLK
  return pl.pallas_call(
      _tc_body,
      grid=(grid,),
      in_specs=[
          pl.BlockSpec((ROW_BLK, D), lambda i: (i, 0)),
          pl.BlockSpec((NC, ROW_BLK, D), lambda i: (0, i, 0)),
          pl.BlockSpec((NC, ROW_BLK, 1), lambda i: (0, i, 0)),
          pl.BlockSpec((D, D), lambda i: (0, 0)),
          pl.BlockSpec((D, D), lambda i: (0, 0)),
          pl.BlockSpec((1, D), lambda i: (0, 0)),
      ],
      out_specs=pl.BlockSpec((ROW_BLK, D), lambda i: (i, 0)),
      out_shape=jax.ShapeDtypeStruct((N, D), jnp.float32),
  )(x, agg, cnt, wl_t, wr_t, b2d)


@jax.jit
def kernel(x, edge_index, W_l, b_l, W_r):
  src = edge_index[0].astype(jnp.int32)
  dst = edge_index[1].astype(jnp.int32)
  pad = E_PAD - E
  src = jnp.concatenate([src, jnp.zeros((pad,), jnp.int32)])
  dst = jnp.concatenate([dst, jnp.full((pad,), N, jnp.int32)])

  z2d = jnp.zeros((ROWS_PER_TILE, D), jnp.float32)
  z1d = jnp.zeros((ROWS_PER_TILE,), jnp.float32)
  ones_h = jnp.ones((CHUNK,), jnp.float32)

  agg, cnt = _sc_aggregate(x, src, dst, z2d, z1d, ones_h)

  out = _tc_tail(x, agg, cnt.reshape(NC, N_PAD, 1),
                 W_l.T, W_r.T, b_l.reshape(1, D))
  return out
